# Initial kernel scaffold; baseline (speedup 1.0000x reference)
import jax
import jax.numpy as jnp
from jax.experimental import pallas as pl


def kernel(active_iter_count_labels, current_iter_mask, full_labels):
    raise NotImplementedError("write your pallas kernel here")


# speedup vs baseline: 1.4879x; 1.4879x over previous
import functools

import jax
import jax.numpy as jnp
from jax import lax
from jax.experimental import pallas as pl
from jax.experimental.pallas import tpu as pltpu
from jax.experimental.pallas import tpu_sc as plsc

_B, _S = 16, 4096
_L = 16
_NBLK = _S // _L
_IGNORE = -100
_NC = 2

_mesh = plsc.VectorSubcoreMesh(core_axis_name="c", subcore_axis_name="s")


@functools.partial(
    pl.kernel,
    mesh=_mesh,
    compiler_params=pltpu.CompilerParams(needs_layout_passes=False),
    out_type=jax.ShapeDtypeStruct((_B, _S), jnp.int32),
    scratch_types=[
        pltpu.VMEM((_S,), jnp.int32),
        pltpu.VMEM((_S,), jnp.int32),
        pltpu.VMEM((_S,), jnp.int32),
    ],
)
def _sc_update(mask_hbm, act_hbm, full_hbm, out_hbm, mask_v, act_v, full_v):
    wid = lax.axis_index("s") * _NC + lax.axis_index("c")

    @pl.when(wid < _B)
    def _():
        pltpu.sync_copy(mask_hbm.at[wid], mask_v)
        pltpu.sync_copy(act_hbm.at[wid], act_v)
        pltpu.sync_copy(full_hbm.at[wid], full_v)

        def body(i, carry):
            m = mask_v[pl.ds(i * _L, _L)]
            cs = plsc.cumsum(m)
            pos = jnp.maximum(cs + (carry - 1), 0)
            g = plsc.load_gather(act_v, [pos])
            keep = (m > 0) & (g != _IGNORE)
            tmp = jnp.where(keep, g, 0)
            f = full_v[pl.ds(i * _L, _L)]
            full_v[pl.ds(i * _L, _L)] = jnp.maximum(f, tmp)
            return carry + jnp.sum(m)

        lax.fori_loop(0, _NBLK, body, jnp.int32(0))
        pltpu.sync_copy(full_v, out_hbm.at[wid])


def kernel(active_iter_count_labels, current_iter_mask, full_labels):
    active = active_iter_count_labels.astype(jnp.int32)
    mask_i32 = current_iter_mask.astype(jnp.int32)
    new_full = _sc_update(mask_i32, active, full_labels)
    return active, new_full


# speedup vs baseline: 1.5279x; 1.0268x over previous
import functools

import jax
import jax.numpy as jnp
from jax import lax
from jax.experimental import pallas as pl
from jax.experimental.pallas import tpu as pltpu
from jax.experimental.pallas import tpu_sc as plsc

_B, _S = 16, 4096
_L = 16
_NBLK = _S // _L
_IGNORE = -100
_NC = 2
_K = 8

_mesh = plsc.VectorSubcoreMesh(core_axis_name="c", subcore_axis_name="s")


@functools.partial(
    pl.kernel,
    mesh=_mesh,
    compiler_params=pltpu.CompilerParams(needs_layout_passes=False),
    out_type=jax.ShapeDtypeStruct((_B, _S), jnp.int32),
    scratch_types=[
        pltpu.VMEM((_S,), jnp.int32),
        pltpu.VMEM((_S,), jnp.int32),
        pltpu.VMEM((_S,), jnp.int32),
        pltpu.SemaphoreType.DMA,
        pltpu.SemaphoreType.DMA,
        pltpu.SemaphoreType.DMA,
    ],
)
def _sc_update(mask_hbm, act_hbm, full_hbm, out_hbm, mask_v, act_v, full_v,
               sem0, sem1, sem2):
    wid = lax.axis_index("s") * _NC + lax.axis_index("c")

    @pl.when(wid < _B)
    def _():
        c0 = pltpu.async_copy(mask_hbm.at[wid], mask_v, sem0)
        c1 = pltpu.async_copy(act_hbm.at[wid], act_v, sem1)
        c2 = pltpu.async_copy(full_hbm.at[wid], full_v, sem2)
        c0.wait()
        c1.wait()
        c2.wait()

        def body(i, carry):
            for k in range(_K):
                base = (i * _K + k) * _L
                m = mask_v[pl.ds(base, _L)]
                mb = m > 0
                cs = plsc.cumsum(m)
                pos = jnp.maximum(cs + (carry - 1), 0)
                g = plsc.load_gather(act_v, [pos])
                keep = mb & (g != _IGNORE)
                tmp = jnp.where(keep, g, 0)
                f = full_v[pl.ds(base, _L)]
                full_v[pl.ds(base, _L)] = jnp.maximum(f, tmp)
                carry = carry + plsc.all_reduce_population_count(mb)
            return carry

        lax.fori_loop(0, _NBLK // _K, body, jnp.zeros((_L,), jnp.int32))
        pltpu.sync_copy(full_v, out_hbm.at[wid])


def kernel(active_iter_count_labels, current_iter_mask, full_labels):
    active = active_iter_count_labels.astype(jnp.int32)
    mask_i32 = current_iter_mask.astype(jnp.int32)
    new_full = _sc_update(mask_i32, active, full_labels)
    return active, new_full
